# baseline (speedup 1.0000x reference)
import functools

import jax
import jax.numpy as jnp
from jax import lax
from jax.experimental import pallas as pl
from jax.experimental.pallas import tpu as pltpu
from jax.experimental.pallas import tpu_sc as plsc

_N = 16384
_D = 32
_NC = 2
_NS = 16
_NW = _NC * _NS
_BPW = _N // _NW
_CH = 128
_NCH = _BPW // _CH
_L = 16


def _partial_sums(idx3, table):
    mesh = plsc.VectorSubcoreMesh(core_axis_name="c", subcore_axis_name="s")

    @functools.partial(
        pl.kernel,
        mesh=mesh,
        out_type=jax.ShapeDtypeStruct((_NW, _D), jnp.float32),
        scratch_types=[
            pltpu.VMEM((_NCH, _CH), jnp.int32),
            pltpu.VMEM((_BPW, _D), jnp.float32),
            pltpu.VMEM((_D,), jnp.float32),
            pltpu.SemaphoreType.DMA,
        ],
        compiler_params=pltpu.CompilerParams(use_tc_tiling_on_sc=False),
    )
    def k(idx_hbm, tab_hbm, out_hbm, idx_v, rows_v, acc_v, sem):
        wid = lax.axis_index("s") * _NC + lax.axis_index("c")
        pltpu.sync_copy(idx_hbm.at[wid], idx_v)
        copies = [
            pltpu.async_copy(
                tab_hbm.at[idx_v.at[j]], rows_v.at[pl.ds(j * _CH, _CH)], sem
            )
            for j in range(_NCH)
        ]
        for c in copies:
            c.wait()

        def body(i, carry):
            a0, a1 = carry
            a0 = a0 + rows_v[i, pl.ds(0, _L)]
            a1 = a1 + rows_v[i, pl.ds(_L, _L)]
            return a0, a1

        a0, a1 = lax.fori_loop(
            0, _BPW, body, (jnp.zeros((_L,), jnp.float32),) * 2
        )
        scale = jnp.float32(1.0 / _N)
        acc_v[pl.ds(0, _L)] = a0 * scale
        acc_v[pl.ds(_L, _L)] = a1 * scale
        pltpu.sync_copy(acc_v, out_hbm.at[wid])

    return k(idx3, table)


@jax.jit
def kernel(action_indices, embedding_weight):
    idx3 = action_indices.astype(jnp.int32).reshape(_NW, _NCH, _CH)
    partials = _partial_sums(idx3, embedding_weight)
    return partials.sum(axis=0)


# speedup vs baseline: 3.2149x; 3.2149x over previous
import functools

import jax
import jax.numpy as jnp
from jax import lax
from jax.experimental import pallas as pl
from jax.experimental.pallas import tpu as pltpu
from jax.experimental.pallas import tpu_sc as plsc

_N = 16384
_V = 1_000_000
_D = 32
_NC = 2
_NS = 16
_NW = _NC * _NS
_L = 16
_BLK = 128
_NB = _V // _BLK
_TAIL = _V - _NB * _BLK
_NBT = -(-_NB // _NW)
_IPT = _N // _NS
_VP = _NBT * _NW * _BLK
_ZPT = _VP // _NS
_ZSEG = _ZPT // 4
_CH = _BLK // _L


def _count_scan(idx16, tab_t, tail_t):
    mesh = plsc.VectorSubcoreMesh(core_axis_name="c", subcore_axis_name="s")

    @functools.partial(
        pl.kernel,
        mesh=mesh,
        out_type=jax.ShapeDtypeStruct((_NW * _D,), jnp.float32),
        scratch_types=[
            pltpu.VMEM_SHARED((_VP,), jnp.float32),
            pltpu.VMEM((_ZSEG,), jnp.float32),
            pltpu.VMEM((_IPT,), jnp.int32),
            pltpu.VMEM((_IPT,), jnp.float32),
            pltpu.VMEM((_NBT * _BLK,), jnp.float32),
            pltpu.VMEM((2, _D, _BLK), jnp.float32),
            pltpu.VMEM((_D, _TAIL), jnp.float32),
            pltpu.VMEM((_D,), jnp.float32),
            pltpu.SemaphoreType.DMA,
            pltpu.SemaphoreType.DMA,
        ],
        compiler_params=pltpu.CompilerParams(needs_layout_passes=False),
    )
    def k(idx_hbm, tab_hbm, tail_hbm, out_hbm, cnt_sh, zero_v, idx_v, ones_v,
          cnt_v, blk_v, tail_v, acc_v, sem0, sem1):
        cid = lax.axis_index("c")
        sid = lax.axis_index("s")
        wid = sid * _NC + cid

        pltpu.sync_copy(idx_hbm.at[pl.ds(sid * _IPT, _IPT)], idx_v)
        zeros16 = jnp.zeros((_L,), jnp.float32)
        ones16 = jnp.ones((_L,), jnp.float32)

        def fill(i, carry):
            zero_v[pl.ds(i * _L, _L)] = zeros16
            return carry

        lax.fori_loop(0, _ZSEG // _L, fill, 0)

        def fill1(i, carry):
            ones_v[pl.ds(i * _L, _L)] = ones16
            return carry

        lax.fori_loop(0, _IPT // _L, fill1, 0)

        zbase = sid * _ZPT
        for j in range(4):
            pltpu.sync_copy(zero_v, cnt_sh.at[pl.ds(zbase + j * _ZSEG, _ZSEG)])
        plsc.subcore_barrier()

        pltpu.sync_copy(ones_v, cnt_sh.at[idx_v], add=True)
        plsc.subcore_barrier()

        base = wid * _NBT
        pltpu.sync_copy(
            cnt_sh.at[pl.ds(base * _BLK, _NBT * _BLK)], cnt_v
        )

        _LAST_VALID = (_NB - (_NW - 1) * _NBT) * _BLK

        @pl.when(wid == _NW - 1)
        def _():
            def zfix(i, carry):
                cnt_v[pl.ds(_LAST_VALID + i * _L, _L)] = zeros16
                return carry

            lax.fori_loop(0, (_NBT * _BLK - _LAST_VALID) // _L, zfix, 0)

        def fire(k_rel, slot, sem):
            blk = jnp.minimum(base + k_rel, _NB - 1)
            off = pl.multiple_of(blk * _BLK, _BLK)
            pltpu.make_async_copy(
                tab_hbm.at[:, pl.ds(off, _BLK)], blk_v.at[slot], sem
            ).start()

        def wait(slot, sem):
            pltpu.make_async_copy(
                tab_hbm.at[:, pl.ds(0, _BLK)], blk_v.at[slot], sem
            ).wait()

        fire(0, 0, sem0)

        def body(kk, accs):
            par = lax.rem(kk, 2)

            @pl.when((kk + 1 < _NBT) & (par == 0))
            def _():
                fire(kk + 1, 1, sem1)

            @pl.when((kk + 1 < _NBT) & (par == 1))
            def _():
                fire(kk + 1, 0, sem0)

            @pl.when(par == 0)
            def _():
                wait(0, sem0)

            @pl.when(par == 1)
            def _():
                wait(1, sem1)

            new = []
            for c in range(_D):
                a = accs[c]
                for ch in range(_CH):
                    cv = cnt_v[pl.ds(kk * _BLK + ch * _L, _L)]
                    bv = blk_v[par, c, pl.ds(ch * _L, _L)]
                    a = a + cv * bv
                new.append(a)
            return tuple(new)

        accs = lax.fori_loop(
            0, _NBT, body,
            tuple(jnp.zeros((_L,), jnp.float32) for _ in range(_D)),
            unroll=False,
        )

        is_last = wid == _NW - 1

        @pl.when(is_last)
        def _():
            pltpu.sync_copy(tail_hbm, tail_v)
            pltpu.sync_copy(
                cnt_sh.at[pl.ds(_NB * _BLK, _TAIL)],
                cnt_v.at[pl.ds(0, _TAIL)],
            )

        tail_accs = []
        for c in range(_D):
            a = accs[c]
            for ch in range(_TAIL // _L):
                cv = cnt_v[pl.ds(ch * _L, _L)]
                tv = tail_v[c, pl.ds(ch * _L, _L)]
                contrib = jnp.where(is_last, cv * tv, zeros16)
                a = a + contrib
            tail_accs.append(a)

        scale = jnp.float32(1.0 / _N)
        iota = lax.iota(jnp.int32, _L)
        for h in range(2):
            out_vec = jnp.zeros((_L,), jnp.float32)
            for j in range(_L):
                s = jnp.sum(tail_accs[h * _L + j]) * scale
                out_vec = jnp.where(iota == j, s, out_vec)
            acc_v[pl.ds(h * _L, _L)] = out_vec
        pltpu.sync_copy(acc_v, out_hbm.at[pl.ds(wid * _D, _D)])

    return k(idx16, tab_t, tail_t)


@jax.jit
def kernel(action_indices, embedding_weight):
    idx_flat = action_indices.astype(jnp.int32)
    tail_t = embedding_weight[_NB * _BLK :].T
    partials = _count_scan(idx_flat, embedding_weight.T, tail_t)
    return partials.reshape(_NW, _D).sum(axis=0)


# speedup vs baseline: 3.5807x; 1.1138x over previous
import functools

import jax
import jax.numpy as jnp
from jax import lax
from jax.experimental import pallas as pl
from jax.experimental.pallas import tpu as pltpu
from jax.experimental.pallas import tpu_sc as plsc

_N = 16384
_V = 1_000_000
_D = 32
_NC = 2
_NS = 16
_NW = _NC * _NS
_L = 16
_BLK = 256
_NB = _V // _BLK
_TAIL = _V - _NB * _BLK
_NBT = -(-_NB // _NW)
_IPT = _N // _NS
_VP = _NBT * _NW * _BLK
_ZPT = _VP // _NS
_ZSEG = _ZPT // 8
_CH = _BLK // _L


def _count_scan(idx16, tab_t, tail_t):
    mesh = plsc.VectorSubcoreMesh(core_axis_name="c", subcore_axis_name="s")

    @functools.partial(
        pl.kernel,
        mesh=mesh,
        out_type=jax.ShapeDtypeStruct((_NW * _D,), jnp.float32),
        scratch_types=[
            pltpu.VMEM_SHARED((_VP,), jnp.float32),
            pltpu.VMEM((_ZSEG,), jnp.float32),
            pltpu.VMEM((_IPT,), jnp.int32),
            pltpu.VMEM((_IPT,), jnp.float32),
            pltpu.VMEM((_NBT * _BLK,), jnp.float32),
            pltpu.VMEM((2, _D, _BLK), jnp.float32),
            pltpu.VMEM((_D, _TAIL), jnp.float32),
            pltpu.VMEM((_D,), jnp.float32),
            pltpu.SemaphoreType.DMA,
            pltpu.SemaphoreType.DMA,
        ],
        compiler_params=pltpu.CompilerParams(needs_layout_passes=False),
    )
    def k(idx_hbm, tab_hbm, tail_hbm, out_hbm, cnt_sh, zero_v, idx_v, ones_v,
          cnt_v, blk_v, tail_v, acc_v, sem0, sem1):
        cid = lax.axis_index("c")
        sid = lax.axis_index("s")
        wid = sid * _NC + cid

        pltpu.sync_copy(idx_hbm.at[pl.ds(sid * _IPT, _IPT)], idx_v)
        zeros16 = jnp.zeros((_L,), jnp.float32)
        ones16 = jnp.ones((_L,), jnp.float32)

        def fill(i, carry):
            zero_v[pl.ds(i * _L, _L)] = zeros16
            return carry

        lax.fori_loop(0, _ZSEG // _L, fill, 0)

        def fill1(i, carry):
            ones_v[pl.ds(i * _L, _L)] = ones16
            return carry

        lax.fori_loop(0, _IPT // _L, fill1, 0)

        zbase = sid * _ZPT
        for j in range(8):
            pltpu.sync_copy(zero_v, cnt_sh.at[pl.ds(zbase + j * _ZSEG, _ZSEG)])
        plsc.subcore_barrier()

        pltpu.sync_copy(ones_v, cnt_sh.at[idx_v], add=True)
        plsc.subcore_barrier()

        base = wid * _NBT
        pltpu.sync_copy(
            cnt_sh.at[pl.ds(base * _BLK, _NBT * _BLK)], cnt_v
        )

        _LAST_VALID = (_NB - (_NW - 1) * _NBT) * _BLK

        @pl.when(wid == _NW - 1)
        def _():
            def zfix(i, carry):
                cnt_v[pl.ds(_LAST_VALID + i * _L, _L)] = zeros16
                return carry

            lax.fori_loop(0, (_NBT * _BLK - _LAST_VALID) // _L, zfix, 0)

        def fire(k_rel, slot, sem):
            blk = jnp.minimum(base + k_rel, _NB - 1)
            off = pl.multiple_of(blk * _BLK, _BLK)
            pltpu.make_async_copy(
                tab_hbm.at[:, pl.ds(off, _BLK)], blk_v.at[slot], sem
            ).start()

        def wait(slot, sem):
            pltpu.make_async_copy(
                tab_hbm.at[:, pl.ds(0, _BLK)], blk_v.at[slot], sem
            ).wait()

        fire(0, 0, sem0)

        def body(kk, accs):
            par = lax.rem(kk, 2)

            @pl.when((kk + 1 < _NBT) & (par == 0))
            def _():
                fire(kk + 1, 1, sem1)

            @pl.when((kk + 1 < _NBT) & (par == 1))
            def _():
                fire(kk + 1, 0, sem0)

            @pl.when(par == 0)
            def _():
                wait(0, sem0)

            @pl.when(par == 1)
            def _():
                wait(1, sem1)

            cvs = [
                cnt_v[pl.ds(kk * _BLK + ch * _L, _L)] for ch in range(_CH)
            ]
            new = []
            for c in range(_D):
                a = accs[c]
                for ch in range(_CH):
                    bv = blk_v[par, c, pl.ds(ch * _L, _L)]
                    a = a + cvs[ch] * bv
                new.append(a)
            return tuple(new)

        accs = lax.fori_loop(
            0, _NBT, body,
            tuple(jnp.zeros((_L,), jnp.float32) for _ in range(_D)),
            unroll=False,
        )

        is_last = wid == _NW - 1

        @pl.when(is_last)
        def _():
            pltpu.sync_copy(tail_hbm, tail_v)
            pltpu.sync_copy(
                cnt_sh.at[pl.ds(_NB * _BLK, _TAIL)],
                cnt_v.at[pl.ds(0, _TAIL)],
            )

        tail_accs = []
        for c in range(_D):
            a = accs[c]
            for ch in range(_TAIL // _L):
                cv = cnt_v[pl.ds(ch * _L, _L)]
                tv = tail_v[c, pl.ds(ch * _L, _L)]
                contrib = jnp.where(is_last, cv * tv, zeros16)
                a = a + contrib
            tail_accs.append(a)

        scale = jnp.float32(1.0 / _N)
        iota = lax.iota(jnp.int32, _L)
        for h in range(2):
            out_vec = jnp.zeros((_L,), jnp.float32)
            for j in range(_L):
                s = jnp.sum(tail_accs[h * _L + j]) * scale
                out_vec = jnp.where(iota == j, s, out_vec)
            acc_v[pl.ds(h * _L, _L)] = out_vec
        pltpu.sync_copy(acc_v, out_hbm.at[pl.ds(wid * _D, _D)])

    return k(idx16, tab_t, tail_t)


@jax.jit
def kernel(action_indices, embedding_weight):
    idx_flat = action_indices.astype(jnp.int32)
    tail_t = embedding_weight[_NB * _BLK :].T
    partials = _count_scan(idx_flat, embedding_weight.T, tail_t)
    return partials.reshape(_NW, _D).sum(axis=0)
